# speedup vs baseline: 1.5493x; 1.5493x over previous
import jax
import jax.numpy as jnp
from jax.experimental import pallas as pl
from jax.experimental.pallas import tpu as pltpu

_ROWS = 112
_VMEM = 60 * 1024 * 1024


def _fused_block_kernel(p_ref, w1_ref, b1_ref, w2_ref, b2_ref, w3_ref,
                        b3_ref, wf1_ref, bf1_ref, wf2_ref, bf2_ref, o_ref):
    m = p_ref.shape[0]
    img = m // _ROWS

    a1 = jnp.dot(p_ref[...], w1_ref[...], preferred_element_type=jnp.float32)
    a1 = jnp.maximum(a1 + b1_ref[...], 0.0).astype(jnp.bfloat16)

    a1p = jnp.concatenate([a1, jnp.zeros((16, 128), jnp.bfloat16)], axis=0)
    x2 = jnp.concatenate([a1p[off:off + m] for off in (0, 1, 10, 11)], axis=1)
    a2 = jnp.dot(x2, w2_ref[...], preferred_element_type=jnp.float32)
    a2 = jnp.maximum(a2 + b2_ref[...], 0.0).astype(jnp.bfloat16)

    a2p = jnp.concatenate([a2, jnp.zeros((24, 64), jnp.bfloat16)], axis=0)
    x3 = jnp.concatenate(
        [a2p[off:off + m] for off in (0, 1, 2, 10, 11, 12, 20, 21, 22)],
        axis=1)
    a3 = jnp.dot(x3, w3_ref[...], preferred_element_type=jnp.float32)
    a3 = jnp.maximum(a3 + b3_ref[...], 0.0).astype(jnp.bfloat16)

    a3r = a3.reshape(img, _ROWS, 64)
    xf = jnp.concatenate(
        [a3r[:, u * 10 + v, :] for u in range(7) for v in range(7)], axis=1)
    h = jnp.dot(xf, wf1_ref[...], preferred_element_type=jnp.float32)
    h = jnp.maximum(h + bf1_ref[...], 0.0)

    logit = jnp.sum(h * wf2_ref[...], axis=1, keepdims=True) + bf2_ref[...]
    o_ref[...] = pl.reciprocal(1.0 + jnp.exp(-logit), approx=True)


def _conv1_patch_rows(x_nchw):
    b, c, h, w = x_nchw.shape
    x = jnp.transpose(x_nchw, (0, 2, 3, 1)).astype(jnp.bfloat16)
    taps = []
    for ki in range(8):
        for kj in range(8):
            taps.append(x[:, ki:ki + 77:4, kj:kj + 77:4, :])
    t = jnp.stack(taps, axis=3).reshape(b, 20, 20, 64 * c)
    t = t.reshape(b, 10, 2, 10, 2, 64 * c)
    t = jnp.transpose(t, (0, 1, 3, 2, 4, 5)).reshape(b, 100, 4 * 64 * c)
    t = jnp.pad(t, ((0, 0), (0, _ROWS - 100), (0, 0)))
    return t.reshape(b * _ROWS, 4 * 64 * c)


def kernel(w1, b1, w2, b2, w3, b3, wf1, bf1, wf2, bf2, x):
    b = x.shape[0]
    img = next(g for g in (32, 16, 8, 4, 2, 1) if b % g == 0)

    patches = _conv1_patch_rows(x)
    w1bd = jnp.kron(jnp.eye(4, dtype=jnp.bfloat16), w1)
    b1t = jnp.tile(b1, (1, 4))
    w2c = w2.reshape(4 * 128, 64)
    w3c = w3.reshape(9 * 64, 64)
    wf1c = wf1.reshape(49 * 64, 512)

    m = img * _ROWS
    out = pl.pallas_call(
        _fused_block_kernel,
        out_shape=jax.ShapeDtypeStruct((b, 1), jnp.float32),
        grid=(b // img,),
        in_specs=[
            pl.BlockSpec((m, 1024), lambda i: (i, 0)),
            pl.BlockSpec((1024, 128), lambda i: (0, 0)),
            pl.BlockSpec((1, 128), lambda i: (0, 0)),
            pl.BlockSpec((512, 64), lambda i: (0, 0)),
            pl.BlockSpec((1, 64), lambda i: (0, 0)),
            pl.BlockSpec((576, 64), lambda i: (0, 0)),
            pl.BlockSpec((1, 64), lambda i: (0, 0)),
            pl.BlockSpec((3136, 512), lambda i: (0, 0)),
            pl.BlockSpec((1, 512), lambda i: (0, 0)),
            pl.BlockSpec((1, 512), lambda i: (0, 0)),
            pl.BlockSpec((1, 1), lambda i: (0, 0)),
        ],
        out_specs=pl.BlockSpec((img, 1), lambda i: (i, 0)),
        compiler_params=pltpu.CompilerParams(
            dimension_semantics=("parallel",),
            vmem_limit_bytes=_VMEM),
    )(patches, w1bd, b1t, w2c, b2, w3c, b3, wf1c, bf1, wf2, bf2)
    return out
